# baseline (speedup 1.0000x reference)
import jax
import jax.numpy as jnp
from jax import lax
from jax.experimental import pallas as pl
from jax.experimental.pallas import tpu as pltpu
from jax.experimental.pallas import tpu_sc as plsc

D = 64
NC, NS = 2, 16
NW = NC * NS
CH = 128


def _make_kernel(B):
    assert B % (NW * CH) == 0
    S = B // (NW * CH)
    mesh = plsc.VectorSubcoreMesh(core_axis_name="c", subcore_axis_name="s")

    def body(idx_hbm, table_hbm, out_hbm, idx_v, rows_v, gsem):
        wid = lax.axis_index("s") * NC + lax.axis_index("c")
        base = wid * (S * CH)
        pltpu.sync_copy(idx_hbm.at[wid], idx_v)

        def step(s, carry):
            pltpu.async_copy(table_hbm.at[idx_v.at[s]], rows_v, gsem).wait()
            pltpu.sync_copy(rows_v, out_hbm.at[pl.ds(base + s * CH, CH)])
            return carry

        lax.fori_loop(0, S, step, 0)

    return pl.kernel(
        body,
        out_type=jax.ShapeDtypeStruct((B, D), jnp.float32),
        mesh=mesh,
        compiler_params=pltpu.CompilerParams(use_tc_tiling_on_sc=False),
        scratch_types=[
            pltpu.VMEM((S, CH), jnp.int32),
            pltpu.VMEM((CH, D), jnp.float32),
            pltpu.SemaphoreType.DMA,
        ],
    )


def kernel(x, table):
    B = x.size
    xf = x.reshape(NW, B // (NW * CH), CH).astype(jnp.int32)
    out = _make_kernel(B)(xf, table)
    return out.reshape(*x.shape, D)


# speedup vs baseline: 1.1157x; 1.1157x over previous
import jax
import jax.numpy as jnp
from jax import lax
from jax.experimental import pallas as pl
from jax.experimental.pallas import tpu as pltpu
from jax.experimental.pallas import tpu_sc as plsc

D = 64
NC, NS = 2, 16
NW = NC * NS
CH = 128


def _make_kernel(B):
    assert B % (NW * CH) == 0
    S = B // (NW * CH)
    mesh = plsc.VectorSubcoreMesh(core_axis_name="c", subcore_axis_name="s")

    NBUF = 8

    def body(idx_hbm, table_hbm, out_hbm, idx_v, rows_v, gsem, ssem):
        wid = lax.axis_index("s") * NC + lax.axis_index("c")
        base = wid * (S * CH)
        pltpu.sync_copy(idx_hbm.at[wid], idx_v)

        def g_copy(t):
            b = lax.rem(t, NBUF)
            return pltpu.make_async_copy(
                table_hbm.at[idx_v.at[t]], rows_v.at[b], gsem.at[b])

        def s_copy(t):
            b = lax.rem(t, NBUF)
            return pltpu.make_async_copy(
                rows_v.at[b], out_hbm.at[pl.ds(base + t * CH, CH)], ssem.at[b])

        def step(t, carry):
            @pl.when(jnp.logical_and(t >= NBUF, t < S + NBUF))
            def _():
                s_copy(t - NBUF).wait()

            @pl.when(t < S)
            def _():
                g_copy(t).start()

            u = t - (NBUF - 1)

            @pl.when(jnp.logical_and(u >= 0, u < S))
            def _():
                g_copy(u).wait()
                s_copy(u).start()

            return carry

        lax.fori_loop(0, S + NBUF - 1, step, 0)
        s_copy(S - 1).wait()

    return pl.kernel(
        body,
        out_type=jax.ShapeDtypeStruct((B, D), jnp.float32),
        mesh=mesh,
        compiler_params=pltpu.CompilerParams(use_tc_tiling_on_sc=False),
        scratch_types=[
            pltpu.VMEM((S, CH), jnp.int32),
            pltpu.VMEM((NBUF, CH, D), jnp.float32),
            pltpu.SemaphoreType.DMA((NBUF,)),
            pltpu.SemaphoreType.DMA((NBUF,)),
        ],
    )


def kernel(x, table):
    B = x.size
    xf = x.reshape(NW, B // (NW * CH), CH).astype(jnp.int32)
    out = _make_kernel(B)(xf, table)
    return out.reshape(*x.shape, D)
